# Initial kernel scaffold; baseline (speedup 1.0000x reference)
import jax
import jax.numpy as jnp
from jax.experimental import pallas as pl


def kernel(x, Wr, br, W1, b1, W2, b2, A_down, bA_down, B_down, bB_down, A_up, bA_up, B_up, bB_up):
    raise NotImplementedError("write your pallas kernel here")


# speedup vs baseline: 2.5234x; 2.5234x over previous
import jax
import jax.numpy as jnp
from jax.experimental import pallas as pl

B, S, D, DFF, E, R = 2, 2048, 1024, 4096, 8, 32
ER = E * R
SCALING = 1.0 / 32.0
TM = 256


def _dot(a, b, dims):
    return jax.lax.dot_general(a, b, (dims, ((), ())),
                               preferred_element_type=jnp.float32)


def _moe_kernel(x_ref, wr_ref, br_ref, w1_ref, b1_ref, w2_ref, b2_ref,
                adn_ref, badn_ref, bdn_ref, bbdn_ref,
                aup_ref, baup_ref, bup_ref, bbup_ref,
                out_ref, routing_ref, ec_ref):
    bf = jnp.bfloat16
    xb = x_ref[...]

    logits = _dot(xb, wr_ref[...], (((1,), (1,)))) + br_ref[...]
    m = jnp.max(logits, axis=-1, keepdims=True)
    ex = jnp.exp(logits - m)
    r = ex / jnp.sum(ex, axis=-1, keepdims=True)
    routing_ref[...] = r
    iot = jax.lax.broadcasted_iota(jnp.int32, (TM, E), 1)
    rmax = jnp.max(r, axis=-1, keepdims=True)
    amin = jnp.min(jnp.where(r == rmax, iot, E), axis=-1, keepdims=True)
    ec_ref[...] = (iot == amin).astype(jnp.float32)

    erow = jax.lax.broadcasted_iota(jnp.int32, (E, ER), 0)
    ecol = jax.lax.broadcasted_iota(jnp.int32, (E, ER), 1)
    expand = (erow == ecol // R).astype(bf)
    rw = _dot(r.astype(bf), expand, (((1,), (0,))))

    h1 = _dot(xb, adn_ref[...], (((1,), (1,)))) + badn_ref[...]
    h1w = (h1 * rw).astype(bf)
    o1 = _dot(xb, w1_ref[...], (((1,), (1,))))
    l1 = _dot(h1w, bdn_ref[...], (((1,), (0,)))) \
        + _dot(r.astype(bf), bbdn_ref[...], (((1,), (0,))))
    down = o1 + b1_ref[...] + SCALING * l1
    a = (0.5 * down * (1.0 + jax.lax.erf(down * 0.7071067811865476))).astype(bf)

    h2 = _dot(a, aup_ref[...], (((1,), (1,)))) + baup_ref[...]
    h2w = (h2 * rw).astype(bf)
    o2 = _dot(a, w2_ref[...], (((1,), (1,))))
    l2 = _dot(h2w, bup_ref[...], (((1,), (0,)))) \
        + _dot(r.astype(bf), bbup_ref[...], (((1,), (0,))))
    out_ref[...] = o2 + b2_ref[...] + SCALING * l2


@jax.jit
def kernel(x, Wr, br, W1, b1, W2, b2, A_down, bA_down, B_down, bB_down,
           A_up, bA_up, B_up, bB_up):
    bf = jnp.bfloat16
    T = B * S
    x2 = x.reshape(T, D).astype(bf)
    adn = A_down.reshape(ER, D).astype(bf)
    bdn = B_down.transpose(0, 2, 1).reshape(ER, DFF).astype(bf)
    aup = A_up.reshape(ER, DFF).astype(bf)
    bup = B_up.transpose(0, 2, 1).reshape(ER, D).astype(bf)
    badn = bA_down.reshape(1, ER)
    baup = bA_up.reshape(1, ER)

    grid = (T // TM,)
    tok = lambda i: (i, 0)
    fixed = lambda i: (0, 0)

    out2, routing2, ec2 = pl.pallas_call(
        _moe_kernel,
        grid=grid,
        in_specs=[
            pl.BlockSpec((TM, D), tok),
            pl.BlockSpec((E, D), fixed),
            pl.BlockSpec((1, E), fixed),
            pl.BlockSpec((DFF, D), fixed),
            pl.BlockSpec((1, DFF), fixed),
            pl.BlockSpec((D, DFF), fixed),
            pl.BlockSpec((1, D), fixed),
            pl.BlockSpec((ER, D), fixed),
            pl.BlockSpec((1, ER), fixed),
            pl.BlockSpec((ER, DFF), fixed),
            pl.BlockSpec((E, DFF), fixed),
            pl.BlockSpec((ER, DFF), fixed),
            pl.BlockSpec((1, ER), fixed),
            pl.BlockSpec((ER, D), fixed),
            pl.BlockSpec((E, D), fixed),
        ],
        out_specs=[
            pl.BlockSpec((TM, D), tok),
            pl.BlockSpec((TM, E), tok),
            pl.BlockSpec((TM, E), tok),
        ],
        out_shape=[
            jax.ShapeDtypeStruct((T, D), jnp.float32),
            jax.ShapeDtypeStruct((T, E), jnp.float32),
            jax.ShapeDtypeStruct((T, E), jnp.float32),
        ],
    )(x2, Wr.astype(bf), br.reshape(1, E), W1.astype(bf), b1.reshape(1, DFF),
      W2.astype(bf), b2.reshape(1, D), adn, badn, bdn, bB_down.astype(bf),
      aup, baup, bup, bB_up.astype(bf))

    return (out2.reshape(B, S, D), routing2.reshape(B, S, E),
            ec2.reshape(B, S, E))
